# Initial kernel scaffold; baseline (speedup 1.0000x reference)
import functools

import jax
import jax.numpy as jnp
from jax import lax
from jax.experimental import pallas as pl
from jax.experimental.pallas import tpu as pltpu
from jax.experimental.pallas import tpu_sc as plsc

N = 10000
E = 320000
F = 128
H = 16
L = 64

NPAD = 10240
NW = 32
EPW = E // NW
CHUNK = 80
NCH = EPW // CHUNK
RPT = NPAD // 16

_MESH = dict(core_axis_name="c", subcore_axis_name="s")


@functools.partial(
    pl.kernel,
    out_type=[
        jax.ShapeDtypeStruct((2, NPAD), jnp.float32),
        jax.ShapeDtypeStruct((NW, NCH, CHUNK), jnp.int32),
    ],
    mesh=plsc.VectorSubcoreMesh(**_MESH),
    scratch_types=[
        pltpu.VMEM((NCH, CHUNK), jnp.int32),
        pltpu.VMEM((NCH, CHUNK), jnp.int32),
        pltpu.VMEM((NCH, CHUNK), jnp.int32),
        pltpu.VMEM((CHUNK,), jnp.float32),
        pltpu.VMEM((RPT,), jnp.float32),
        pltpu.VMEM_SHARED((NPAD,), jnp.float32),
    ],
)
def _deg_adj(frm_hbm, to_hbm, deg_hbm, adj_hbm,
             frm_v, to_v, adj_v, ones_v, zero_v, acc_sh):
    cid = lax.axis_index("c")
    sid = lax.axis_index("s")
    wid = cid * 16 + sid
    z16 = jnp.zeros((16,), jnp.float32)
    o16 = jnp.ones((16,), jnp.float32)

    def fill_z(i, _):
        zero_v[pl.ds(i * 16, 16)] = z16
        return 0
    lax.fori_loop(0, RPT // 16, fill_z, 0)

    def fill_o(i, _):
        ones_v[pl.ds(i * 16, 16)] = o16
        return 0
    lax.fori_loop(0, CHUNK // 16, fill_o, 0)

    pltpu.sync_copy(zero_v, acc_sh.at[pl.ds(sid * RPT, RPT)])
    pltpu.sync_copy(frm_hbm.at[wid], frm_v)
    pltpu.sync_copy(to_hbm.at[wid], to_v)

    def chunk(j, _):
        dump = N + ((wid * NCH + j) * 7) % (NPAD - N)

        def vec(k, _):
            f16 = frm_v[j, pl.ds(k * 16, 16)]
            t16 = to_v[j, pl.ds(k * 16, 16)]
            adj_v[j, pl.ds(k * 16, 16)] = jnp.where(f16 != t16, t16, dump)
            return 0
        lax.fori_loop(0, CHUNK // 16, vec, 0)
        return 0
    lax.fori_loop(0, NCH, chunk, 0)

    plsc.subcore_barrier()

    def scat(j, _):
        pltpu.sync_copy(ones_v, acc_sh.at[adj_v.at[j]], add=True)
        return 0
    lax.fori_loop(0, NCH, scat, 0)

    pltpu.sync_copy(adj_v, adj_hbm.at[wid])
    plsc.subcore_barrier()
    pltpu.sync_copy(acc_sh.at[pl.ds(sid * RPT, RPT)],
                    deg_hbm.at[cid, pl.ds(sid * RPT, RPT)])


def _make_agg(D):
    @functools.partial(
        pl.kernel,
        out_type=jax.ShapeDtypeStruct((2, NPAD, D), jnp.float32),
        mesh=plsc.VectorSubcoreMesh(**_MESH),
        scratch_types=[
            pltpu.VMEM((NCH, CHUNK), jnp.int32),
            pltpu.VMEM((NCH, CHUNK), jnp.int32),
            pltpu.VMEM((CHUNK, D), jnp.float32),
            pltpu.VMEM((CHUNK, D), jnp.float32),
            pltpu.VMEM((RPT, D), jnp.float32),
            pltpu.VMEM_SHARED((NPAD, D), jnp.float32),
            pltpu.SemaphoreType.DMA,
            pltpu.SemaphoreType.DMA,
        ],
    )
    def agg(y_hbm, frm_hbm, adj_hbm, out_hbm,
            frm_v, adj_v, buf0, buf1, zero_v, acc_sh, sem0, sem1):
        cid = lax.axis_index("c")
        sid = lax.axis_index("s")
        wid = cid * 16 + sid
        z16 = jnp.zeros((16,), jnp.float32)

        def zrow(i, _):
            def zcol(k, _):
                zero_v[i, pl.ds(k * 16, 16)] = z16
                return 0
            lax.fori_loop(0, D // 16, zcol, 0)
            return 0
        lax.fori_loop(0, RPT, zrow, 0)

        pltpu.sync_copy(zero_v, acc_sh.at[pl.ds(sid * RPT, RPT)])
        pltpu.sync_copy(frm_hbm.at[wid], frm_v)
        pltpu.sync_copy(adj_hbm.at[wid], adj_v)
        plsc.subcore_barrier()

        pltpu.async_copy(y_hbm.at[frm_v.at[0]], buf0, sem0)

        def pair(p, _):
            j0 = 2 * p
            pltpu.make_async_copy(y_hbm.at[frm_v.at[j0]], buf0, sem0).wait()
            pltpu.async_copy(y_hbm.at[frm_v.at[j0 + 1]], buf1, sem1)
            pltpu.sync_copy(buf0, acc_sh.at[adj_v.at[j0]], add=True)
            pltpu.make_async_copy(y_hbm.at[frm_v.at[j0 + 1]], buf1, sem1).wait()
            pltpu.async_copy(y_hbm.at[frm_v.at[j0 + 2]], buf0, sem0)
            pltpu.sync_copy(buf1, acc_sh.at[adj_v.at[j0 + 1]], add=True)
            return 0
        lax.fori_loop(0, (NCH - 1) // 2, pair, 0)

        pltpu.make_async_copy(y_hbm.at[frm_v.at[NCH - 1]], buf0, sem0).wait()
        pltpu.sync_copy(buf0, acc_sh.at[adj_v.at[NCH - 1]], add=True)

        plsc.subcore_barrier()
        pltpu.sync_copy(acc_sh.at[pl.ds(sid * RPT, RPT)],
                        out_hbm.at[cid, pl.ds(sid * RPT, RPT)])

    return agg


_agg_h = _make_agg(H)
_agg_l = _make_agg(L)


BLK = 1000


def _tc1_body(deg_ref, x_ref, w_ref, dis_ref, y_ref):
    deg = deg_ref[0, :] + deg_ref[1, :] + 1.0
    dis = lax.rsqrt(deg)
    y = jnp.dot(x_ref[...], w_ref[...], preferred_element_type=jnp.float32)
    dis_ref[...] = dis[:, None]
    y_ref[...] = y * dis[:, None]


def _tc2_body(dis_ref, p_ref, y1_ref, b1_ref, w2_ref, y2_ref):
    agg = p_ref[0] + p_ref[1] + y1_ref[...]
    h = jnp.maximum(agg * dis_ref[...] + b1_ref[...], 0.0)
    y2_ref[...] = jnp.dot(h, w2_ref[...],
                          preferred_element_type=jnp.float32) * dis_ref[...]


def _tc3_body(dis_ref, q_ref, y2_ref, b2_ref, out_ref):
    logits = (q_ref[0] + q_ref[1] + y2_ref[...]) * dis_ref[...] + b2_ref[...]
    m = jnp.max(logits, axis=1, keepdims=True)
    s = jnp.log(jnp.sum(jnp.exp(logits - m), axis=1, keepdims=True))
    out_ref[...] = logits - m - s


def _tc1(deg2, x, w1t):
    return pl.pallas_call(
        _tc1_body,
        grid=(N // BLK,),
        in_specs=[
            pl.BlockSpec((2, BLK), lambda i: (0, i)),
            pl.BlockSpec((BLK, F), lambda i: (i, 0)),
            pl.BlockSpec((F, H), lambda i: (0, 0)),
        ],
        out_specs=[
            pl.BlockSpec((BLK, 1), lambda i: (i, 0)),
            pl.BlockSpec((BLK, H), lambda i: (i, 0)),
        ],
        out_shape=[
            jax.ShapeDtypeStruct((N, 1), jnp.float32),
            jax.ShapeDtypeStruct((N, H), jnp.float32),
        ],
    )(deg2, x, w1t)


def _tc2(dis, p1, y1, b1, w2t):
    return pl.pallas_call(
        _tc2_body,
        grid=(N // BLK,),
        in_specs=[
            pl.BlockSpec((BLK, 1), lambda i: (i, 0)),
            pl.BlockSpec((2, BLK, H), lambda i: (0, i, 0)),
            pl.BlockSpec((BLK, H), lambda i: (i, 0)),
            pl.BlockSpec((1, H), lambda i: (0, 0)),
            pl.BlockSpec((H, L), lambda i: (0, 0)),
        ],
        out_specs=pl.BlockSpec((BLK, L), lambda i: (i, 0)),
        out_shape=jax.ShapeDtypeStruct((N, L), jnp.float32),
    )(dis, p1, y1, b1, w2t)


def _tc3(dis, p2, y2, b2):
    return pl.pallas_call(
        _tc3_body,
        grid=(N // BLK,),
        in_specs=[
            pl.BlockSpec((BLK, 1), lambda i: (i, 0)),
            pl.BlockSpec((2, BLK, L), lambda i: (0, i, 0)),
            pl.BlockSpec((BLK, L), lambda i: (i, 0)),
            pl.BlockSpec((1, L), lambda i: (0, 0)),
        ],
        out_specs=pl.BlockSpec((BLK, L), lambda i: (i, 0)),
        out_shape=jax.ShapeDtypeStruct((N, L), jnp.float32),
    )(dis, p2, y2, b2)


def kernel(x, edge_index, W1, b1, W2, b2):
    ei = edge_index.astype(jnp.int32)
    frm = ei[0].reshape(NW, NCH, CHUNK)
    to = ei[1].reshape(NW, NCH, CHUNK)

    deg_p, adj = _deg_adj(frm, to)
    dis, y1 = _tc1(deg_p[:, :N], x, W1.T)
    p1 = _agg_h(y1, frm, adj)
    y2 = _tc2(dis, p1[:, :N], y1, b1.reshape(1, H), W2.T)
    p2 = _agg_l(y2, frm, adj)
    return _tc3(dis, p2[:, :N], y2, b2.reshape(1, L))


# speedup vs baseline: 28.5689x; 28.5689x over previous
import functools

import jax
import jax.numpy as jnp
from jax import lax
from jax.experimental import pallas as pl
from jax.experimental.pallas import tpu as pltpu
from jax.experimental.pallas import tpu_sc as plsc

N = 10000
E = 320000
F = 128
H = 16
L = 64

NPAD = 10240
NW = 32
EPW = E // NW
CHUNK = 80
NCH = EPW // CHUNK
RPT = NPAD // 16

_MESH = dict(core_axis_name="c", subcore_axis_name="s")


@functools.partial(
    pl.kernel,
    out_type=[
        jax.ShapeDtypeStruct((2, NPAD), jnp.float32),
        jax.ShapeDtypeStruct((NW, NCH, CHUNK), jnp.int32),
    ],
    mesh=plsc.VectorSubcoreMesh(**_MESH),
    compiler_params=pltpu.CompilerParams(use_tc_tiling_on_sc=False),
    scratch_types=[
        pltpu.VMEM((NCH, CHUNK), jnp.int32),
        pltpu.VMEM((NCH, CHUNK), jnp.int32),
        pltpu.VMEM((NCH, CHUNK), jnp.int32),
        pltpu.VMEM((CHUNK,), jnp.float32),
        pltpu.VMEM((RPT,), jnp.float32),
        pltpu.VMEM_SHARED((NPAD,), jnp.float32),
    ],
)
def _deg_adj(frm_hbm, to_hbm, deg_hbm, adj_hbm,
             frm_v, to_v, adj_v, ones_v, zero_v, acc_sh):
    cid = lax.axis_index("c")
    sid = lax.axis_index("s")
    wid = cid * 16 + sid
    z16 = jnp.zeros((16,), jnp.float32)
    o16 = jnp.ones((16,), jnp.float32)

    def fill_z(i, _):
        zero_v[pl.ds(i * 16, 16)] = z16
        return 0
    lax.fori_loop(0, RPT // 16, fill_z, 0)

    def fill_o(i, _):
        ones_v[pl.ds(i * 16, 16)] = o16
        return 0
    lax.fori_loop(0, CHUNK // 16, fill_o, 0)

    pltpu.sync_copy(zero_v, acc_sh.at[pl.ds(sid * RPT, RPT)])
    pltpu.sync_copy(frm_hbm.at[wid], frm_v)
    pltpu.sync_copy(to_hbm.at[wid], to_v)

    def chunk(j, _):
        dump = N + ((wid * NCH + j) * 7) % (NPAD - N)

        def vec(k, _):
            f16 = frm_v[j, pl.ds(k * 16, 16)]
            t16 = to_v[j, pl.ds(k * 16, 16)]
            adj_v[j, pl.ds(k * 16, 16)] = jnp.where(f16 != t16, t16, dump)
            return 0
        lax.fori_loop(0, CHUNK // 16, vec, 0)
        return 0
    lax.fori_loop(0, NCH, chunk, 0)

    plsc.subcore_barrier()

    def scat(j, _):
        pltpu.sync_copy(ones_v, acc_sh.at[adj_v.at[j]], add=True)
        return 0
    lax.fori_loop(0, NCH, scat, 0)

    pltpu.sync_copy(adj_v, adj_hbm.at[wid])
    plsc.subcore_barrier()
    pltpu.sync_copy(acc_sh.at[pl.ds(sid * RPT, RPT)],
                    deg_hbm.at[cid, pl.ds(sid * RPT, RPT)])


def _make_agg(D):
    @functools.partial(
        pl.kernel,
        out_type=jax.ShapeDtypeStruct((2, NPAD, D), jnp.float32),
        mesh=plsc.VectorSubcoreMesh(**_MESH),
        compiler_params=pltpu.CompilerParams(use_tc_tiling_on_sc=False),
        scratch_types=[
            pltpu.VMEM((NCH, CHUNK), jnp.int32),
            pltpu.VMEM((NCH, CHUNK), jnp.int32),
            pltpu.VMEM((CHUNK, D), jnp.float32),
            pltpu.VMEM((CHUNK, D), jnp.float32),
            pltpu.VMEM((RPT, D), jnp.float32),
            pltpu.VMEM_SHARED((NPAD, D), jnp.float32),
            pltpu.SemaphoreType.DMA,
            pltpu.SemaphoreType.DMA,
        ],
    )
    def agg(y_hbm, frm_hbm, adj_hbm, out_hbm,
            frm_v, adj_v, buf0, buf1, zero_v, acc_sh, sem0, sem1):
        cid = lax.axis_index("c")
        sid = lax.axis_index("s")
        wid = cid * 16 + sid
        z16 = jnp.zeros((16,), jnp.float32)

        def zrow(i, _):
            def zcol(k, _):
                zero_v[i, pl.ds(k * 16, 16)] = z16
                return 0
            lax.fori_loop(0, D // 16, zcol, 0)
            return 0
        lax.fori_loop(0, RPT, zrow, 0)

        pltpu.sync_copy(zero_v, acc_sh.at[pl.ds(sid * RPT, RPT)])
        pltpu.sync_copy(frm_hbm.at[wid], frm_v)
        pltpu.sync_copy(adj_hbm.at[wid], adj_v)
        plsc.subcore_barrier()

        pltpu.async_copy(y_hbm.at[frm_v.at[0]], buf0, sem0)

        def pair(p, _):
            j0 = 2 * p
            pltpu.make_async_copy(y_hbm.at[frm_v.at[j0]], buf0, sem0).wait()
            pltpu.async_copy(y_hbm.at[frm_v.at[j0 + 1]], buf1, sem1)
            pltpu.sync_copy(buf0, acc_sh.at[adj_v.at[j0]], add=True)
            pltpu.make_async_copy(y_hbm.at[frm_v.at[j0 + 1]], buf1, sem1).wait()
            pltpu.async_copy(y_hbm.at[frm_v.at[j0 + 2]], buf0, sem0)
            pltpu.sync_copy(buf1, acc_sh.at[adj_v.at[j0 + 1]], add=True)
            return 0
        lax.fori_loop(0, (NCH - 1) // 2, pair, 0)

        pltpu.make_async_copy(y_hbm.at[frm_v.at[NCH - 1]], buf0, sem0).wait()
        pltpu.sync_copy(buf0, acc_sh.at[adj_v.at[NCH - 1]], add=True)

        plsc.subcore_barrier()
        pltpu.sync_copy(acc_sh.at[pl.ds(sid * RPT, RPT)],
                        out_hbm.at[cid, pl.ds(sid * RPT, RPT)])

    return agg


_agg_h = _make_agg(H)
_agg_l = _make_agg(L)


BLK = 1000


def _tc1_body(deg_ref, x_ref, w_ref, dis_ref, y_ref):
    deg = deg_ref[:, 0] + deg_ref[:, 1] + 1.0
    dis = lax.rsqrt(deg)
    y = jnp.dot(x_ref[...], w_ref[...], preferred_element_type=jnp.float32)
    dis_ref[...] = dis[:, None]
    y_ref[...] = y * dis[:, None]


def _tc2_body(dis_ref, p_ref, y1_ref, b1_ref, w2_ref, y2_ref):
    agg = p_ref[0] + p_ref[1] + y1_ref[...]
    h = jnp.maximum(agg * dis_ref[...] + b1_ref[...], 0.0)
    y2_ref[...] = jnp.dot(h, w2_ref[...],
                          preferred_element_type=jnp.float32) * dis_ref[...]


def _tc3_body(dis_ref, q_ref, y2_ref, b2_ref, out_ref):
    logits = (q_ref[0] + q_ref[1] + y2_ref[...]) * dis_ref[...] + b2_ref[...]
    m = jnp.max(logits, axis=1, keepdims=True)
    s = jnp.log(jnp.sum(jnp.exp(logits - m), axis=1, keepdims=True))
    out_ref[...] = logits - m - s


def _tc1(deg2, x, w1t):
    return pl.pallas_call(
        _tc1_body,
        grid=(N // BLK,),
        in_specs=[
            pl.BlockSpec((BLK, 2), lambda i: (i, 0)),
            pl.BlockSpec((BLK, F), lambda i: (i, 0)),
            pl.BlockSpec((F, H), lambda i: (0, 0)),
        ],
        out_specs=[
            pl.BlockSpec((BLK, 1), lambda i: (i, 0)),
            pl.BlockSpec((BLK, H), lambda i: (i, 0)),
        ],
        out_shape=[
            jax.ShapeDtypeStruct((N, 1), jnp.float32),
            jax.ShapeDtypeStruct((N, H), jnp.float32),
        ],
    )(deg2, x, w1t)


def _tc2(dis, p1, y1, b1, w2t):
    return pl.pallas_call(
        _tc2_body,
        grid=(N // BLK,),
        in_specs=[
            pl.BlockSpec((BLK, 1), lambda i: (i, 0)),
            pl.BlockSpec((2, BLK, H), lambda i: (0, i, 0)),
            pl.BlockSpec((BLK, H), lambda i: (i, 0)),
            pl.BlockSpec((1, H), lambda i: (0, 0)),
            pl.BlockSpec((H, L), lambda i: (0, 0)),
        ],
        out_specs=pl.BlockSpec((BLK, L), lambda i: (i, 0)),
        out_shape=jax.ShapeDtypeStruct((N, L), jnp.float32),
    )(dis, p1, y1, b1, w2t)


def _tc3(dis, p2, y2, b2):
    return pl.pallas_call(
        _tc3_body,
        grid=(N // BLK,),
        in_specs=[
            pl.BlockSpec((BLK, 1), lambda i: (i, 0)),
            pl.BlockSpec((2, BLK, L), lambda i: (0, i, 0)),
            pl.BlockSpec((BLK, L), lambda i: (i, 0)),
            pl.BlockSpec((1, L), lambda i: (0, 0)),
        ],
        out_specs=pl.BlockSpec((BLK, L), lambda i: (i, 0)),
        out_shape=jax.ShapeDtypeStruct((N, L), jnp.float32),
    )(dis, p2, y2, b2)


def kernel(x, edge_index, W1, b1, W2, b2):
    ei = edge_index.astype(jnp.int32)
    frm = ei[0].reshape(NW, NCH, CHUNK)
    to = ei[1].reshape(NW, NCH, CHUNK)

    deg_p, adj = _deg_adj(frm, to)
    dis, y1 = _tc1(deg_p[:, :N].T, x, W1.T)
    p1 = _agg_h(y1, frm, adj)
    y2 = _tc2(dis, p1[:, :N], y1, b1.reshape(1, H), W2.T)
    p2 = _agg_l(y2, frm, adj)
    return _tc3(dis, p2[:, :N], y2, b2.reshape(1, L))


# speedup vs baseline: 45.6153x; 1.5967x over previous
import functools

import jax
import jax.numpy as jnp
from jax import lax
from jax.experimental import pallas as pl
from jax.experimental.pallas import tpu as pltpu
from jax.experimental.pallas import tpu_sc as plsc

N = 10000
E = 320000
F = 128
H = 16
L = 64

NPAD = 10240
NW = 32
EPW = E // NW
CHUNK = 80
NCH = EPW // CHUNK
RPT = NPAD // 16

_MESH = dict(core_axis_name="c", subcore_axis_name="s")


@functools.partial(
    pl.kernel,
    out_type=[
        jax.ShapeDtypeStruct((2, NPAD), jnp.float32),
        jax.ShapeDtypeStruct((NW, NCH, CHUNK), jnp.int32),
    ],
    mesh=plsc.VectorSubcoreMesh(**_MESH),
    compiler_params=pltpu.CompilerParams(use_tc_tiling_on_sc=False),
    scratch_types=[
        pltpu.VMEM((NCH, CHUNK), jnp.int32),
        pltpu.VMEM((NCH, CHUNK), jnp.int32),
        pltpu.VMEM((NCH, CHUNK), jnp.int32),
        pltpu.VMEM((CHUNK,), jnp.float32),
        pltpu.VMEM((RPT,), jnp.float32),
        pltpu.VMEM_SHARED((NPAD,), jnp.float32),
    ],
)
def _deg_adj(frm_hbm, to_hbm, deg_hbm, adj_hbm,
             frm_v, to_v, adj_v, ones_v, zero_v, acc_sh):
    cid = lax.axis_index("c")
    sid = lax.axis_index("s")
    wid = cid * 16 + sid
    z16 = jnp.zeros((16,), jnp.float32)
    o16 = jnp.ones((16,), jnp.float32)

    def fill_z(i, _):
        zero_v[pl.ds(i * 16, 16)] = z16
        return 0
    lax.fori_loop(0, RPT // 16, fill_z, 0)

    def fill_o(i, _):
        ones_v[pl.ds(i * 16, 16)] = o16
        return 0
    lax.fori_loop(0, CHUNK // 16, fill_o, 0)

    pltpu.sync_copy(zero_v, acc_sh.at[pl.ds(sid * RPT, RPT)])
    pltpu.sync_copy(frm_hbm.at[wid], frm_v)
    pltpu.sync_copy(to_hbm.at[wid], to_v)

    def chunk(j, _):
        dump = N + ((wid * NCH + j) * 7) % (NPAD - N)

        def vec(k, _):
            f16 = frm_v[j, pl.ds(k * 16, 16)]
            t16 = to_v[j, pl.ds(k * 16, 16)]
            adj_v[j, pl.ds(k * 16, 16)] = jnp.where(f16 != t16, t16, dump)
            return 0
        lax.fori_loop(0, CHUNK // 16, vec, 0)
        return 0
    lax.fori_loop(0, NCH, chunk, 0)

    plsc.subcore_barrier()

    def scat(j, _):
        pltpu.sync_copy(ones_v, acc_sh.at[adj_v.at[j]], add=True)
        return 0
    lax.fori_loop(0, NCH, scat, 0)

    pltpu.sync_copy(adj_v, adj_hbm.at[wid])
    plsc.subcore_barrier()
    pltpu.sync_copy(acc_sh.at[pl.ds(sid * RPT, RPT)],
                    deg_hbm.at[cid, pl.ds(sid * RPT, RPT)])


NBUF = 5
NRING = NCH // NBUF
ZR = 40


def _make_agg(D):
    @functools.partial(
        pl.kernel,
        out_type=jax.ShapeDtypeStruct((2, NPAD, D), jnp.float32),
        mesh=plsc.VectorSubcoreMesh(**_MESH),
        compiler_params=pltpu.CompilerParams(use_tc_tiling_on_sc=False),
        scratch_types=[
            pltpu.VMEM((NCH, CHUNK), jnp.int32),
            pltpu.VMEM((NCH, CHUNK), jnp.int32),
            pltpu.VMEM((NBUF, CHUNK, D), jnp.float32),
            pltpu.VMEM((ZR, D), jnp.float32),
            pltpu.VMEM_SHARED((NPAD, D), jnp.float32),
        ] + [pltpu.SemaphoreType.DMA] * NBUF,
    )
    def agg(y_hbm, frm_hbm, adj_hbm, out_hbm,
            frm_v, adj_v, bufs, zero_v, acc_sh, *sems):
        cid = lax.axis_index("c")
        sid = lax.axis_index("s")
        wid = cid * 16 + sid
        z16 = jnp.zeros((16,), jnp.float32)

        def zrow(i, _):
            def zcol(k, _):
                zero_v[i, pl.ds(k * 16, 16)] = z16
                return 0
            lax.fori_loop(0, D // 16, zcol, 0)
            return 0
        lax.fori_loop(0, ZR, zrow, 0)

        for t in range(RPT // ZR):
            pltpu.sync_copy(zero_v, acc_sh.at[pl.ds(sid * RPT + t * ZR, ZR)])
        pltpu.sync_copy(frm_hbm.at[wid], frm_v)
        pltpu.sync_copy(adj_hbm.at[wid], adj_v)
        plsc.subcore_barrier()

        for b in range(NBUF):
            pltpu.async_copy(y_hbm.at[frm_v.at[b]], bufs.at[b], sems[b])

        def ring(g, _):
            j0 = g * NBUF
            for b in range(NBUF):
                pltpu.make_async_copy(
                    y_hbm.at[frm_v.at[j0 + b]], bufs.at[b], sems[b]).wait()
                pltpu.sync_copy(bufs.at[b], acc_sh.at[adj_v.at[j0 + b]],
                                add=True)
                pltpu.async_copy(
                    y_hbm.at[frm_v.at[j0 + NBUF + b]], bufs.at[b], sems[b])
            return 0
        lax.fori_loop(0, NRING - 1, ring, 0)

        j0 = (NRING - 1) * NBUF
        for b in range(NBUF):
            pltpu.make_async_copy(
                y_hbm.at[frm_v.at[j0 + b]], bufs.at[b], sems[b]).wait()
            pltpu.sync_copy(bufs.at[b], acc_sh.at[adj_v.at[j0 + b]], add=True)

        plsc.subcore_barrier()
        pltpu.sync_copy(acc_sh.at[pl.ds(sid * RPT, RPT)],
                        out_hbm.at[cid, pl.ds(sid * RPT, RPT)])

    return agg


_agg_h = _make_agg(H)
_agg_l = _make_agg(L)


BLK = 1000


def _tc1_body(deg_ref, x_ref, w_ref, dis_ref, y_ref):
    deg = deg_ref[:, 0] + deg_ref[:, 1] + 1.0
    dis = lax.rsqrt(deg)
    y = jnp.dot(x_ref[...], w_ref[...], preferred_element_type=jnp.float32)
    dis_ref[...] = dis[:, None]
    y_ref[...] = y * dis[:, None]


def _tc2_body(dis_ref, p_ref, y1_ref, b1_ref, w2_ref, y2_ref):
    agg = p_ref[0] + p_ref[1] + y1_ref[...]
    h = jnp.maximum(agg * dis_ref[...] + b1_ref[...], 0.0)
    y2_ref[...] = jnp.dot(h, w2_ref[...],
                          preferred_element_type=jnp.float32) * dis_ref[...]


def _tc3_body(dis_ref, q_ref, y2_ref, b2_ref, out_ref):
    logits = (q_ref[0] + q_ref[1] + y2_ref[...]) * dis_ref[...] + b2_ref[...]
    m = jnp.max(logits, axis=1, keepdims=True)
    s = jnp.log(jnp.sum(jnp.exp(logits - m), axis=1, keepdims=True))
    out_ref[...] = logits - m - s


def _tc1(deg2, x, w1t):
    return pl.pallas_call(
        _tc1_body,
        grid=(N // BLK,),
        in_specs=[
            pl.BlockSpec((BLK, 2), lambda i: (i, 0)),
            pl.BlockSpec((BLK, F), lambda i: (i, 0)),
            pl.BlockSpec((F, H), lambda i: (0, 0)),
        ],
        out_specs=[
            pl.BlockSpec((BLK, 1), lambda i: (i, 0)),
            pl.BlockSpec((BLK, H), lambda i: (i, 0)),
        ],
        out_shape=[
            jax.ShapeDtypeStruct((N, 1), jnp.float32),
            jax.ShapeDtypeStruct((N, H), jnp.float32),
        ],
    )(deg2, x, w1t)


def _tc2(dis, p1, y1, b1, w2t):
    return pl.pallas_call(
        _tc2_body,
        grid=(N // BLK,),
        in_specs=[
            pl.BlockSpec((BLK, 1), lambda i: (i, 0)),
            pl.BlockSpec((2, BLK, H), lambda i: (0, i, 0)),
            pl.BlockSpec((BLK, H), lambda i: (i, 0)),
            pl.BlockSpec((1, H), lambda i: (0, 0)),
            pl.BlockSpec((H, L), lambda i: (0, 0)),
        ],
        out_specs=pl.BlockSpec((BLK, L), lambda i: (i, 0)),
        out_shape=jax.ShapeDtypeStruct((N, L), jnp.float32),
    )(dis, p1, y1, b1, w2t)


def _tc3(dis, p2, y2, b2):
    return pl.pallas_call(
        _tc3_body,
        grid=(N // BLK,),
        in_specs=[
            pl.BlockSpec((BLK, 1), lambda i: (i, 0)),
            pl.BlockSpec((2, BLK, L), lambda i: (0, i, 0)),
            pl.BlockSpec((BLK, L), lambda i: (i, 0)),
            pl.BlockSpec((1, L), lambda i: (0, 0)),
        ],
        out_specs=pl.BlockSpec((BLK, L), lambda i: (i, 0)),
        out_shape=jax.ShapeDtypeStruct((N, L), jnp.float32),
    )(dis, p2, y2, b2)


def kernel(x, edge_index, W1, b1, W2, b2):
    ei = edge_index.astype(jnp.int32)
    frm = ei[0].reshape(NW, NCH, CHUNK)
    to = ei[1].reshape(NW, NCH, CHUNK)

    deg_p, adj = _deg_adj(frm, to)
    dis, y1 = _tc1(deg_p[:, :N].T, x, W1.T)
    p1 = _agg_h(y1, frm, adj)
    y2 = _tc2(dis, p1[:, :N], y1, b1.reshape(1, H), W2.T)
    p2 = _agg_l(y2, frm, adj)
    return _tc3(dis, p2[:, :N], y2, b2.reshape(1, L))


# speedup vs baseline: 57.9618x; 1.2707x over previous
import functools

import jax
import jax.numpy as jnp
from jax import lax
from jax.experimental import pallas as pl
from jax.experimental.pallas import tpu as pltpu
from jax.experimental.pallas import tpu_sc as plsc

N = 10000
E = 320000
F = 128
H = 16
L = 64

NPAD = 10240
NW = 32
EPW = E // NW
CHUNK = 80
NCH = EPW // CHUNK
RPT = NPAD // 16
NBUF = 5
NRING = NCH // NBUF
ZR = 40
TAIL = N - 15 * RPT

_MESH = dict(core_axis_name="c", subcore_axis_name="s")


@functools.partial(
    pl.kernel,
    out_type=[
        jax.ShapeDtypeStruct((2, N), jnp.float32),
        jax.ShapeDtypeStruct((NW, NCH, CHUNK), jnp.int32),
    ],
    mesh=plsc.VectorSubcoreMesh(**_MESH),
    compiler_params=pltpu.CompilerParams(use_tc_tiling_on_sc=False),
    scratch_types=[
        pltpu.VMEM((EPW,), jnp.int32),
        pltpu.VMEM((EPW,), jnp.int32),
        pltpu.VMEM((NCH, CHUNK), jnp.int32),
        pltpu.VMEM((CHUNK,), jnp.float32),
        pltpu.VMEM((RPT,), jnp.float32),
        pltpu.VMEM_SHARED((NPAD,), jnp.float32),
    ],
)
def _deg_adj(ei_hbm, deg_hbm, adj_hbm,
             frm_v, to_v, adj_v, ones_v, zero_v, acc_sh):
    cid = lax.axis_index("c")
    sid = lax.axis_index("s")
    wid = cid * 16 + sid
    z16 = jnp.zeros((16,), jnp.float32)
    o16 = jnp.ones((16,), jnp.float32)

    def fill_z(i, _):
        zero_v[pl.ds(i * 16, 16)] = z16
        return 0
    lax.fori_loop(0, RPT // 16, fill_z, 0)

    def fill_o(i, _):
        ones_v[pl.ds(i * 16, 16)] = o16
        return 0
    lax.fori_loop(0, CHUNK // 16, fill_o, 0)

    pltpu.sync_copy(zero_v, acc_sh.at[pl.ds(sid * RPT, RPT)])
    pltpu.sync_copy(ei_hbm.at[0, pl.ds(wid * EPW, EPW)], frm_v)
    pltpu.sync_copy(ei_hbm.at[1, pl.ds(wid * EPW, EPW)], to_v)

    def chunk(j, _):
        dump = N + ((wid * NCH + j) * 7) % (NPAD - N)

        def vec(k, _):
            f16 = frm_v[pl.ds(j * CHUNK + k * 16, 16)]
            t16 = to_v[pl.ds(j * CHUNK + k * 16, 16)]
            adj_v[j, pl.ds(k * 16, 16)] = jnp.where(f16 != t16, t16, dump)
            return 0
        lax.fori_loop(0, CHUNK // 16, vec, 0)
        return 0
    lax.fori_loop(0, NCH, chunk, 0)

    plsc.subcore_barrier()

    def scat(j, _):
        pltpu.sync_copy(ones_v, acc_sh.at[adj_v.at[j]], add=True)
        return 0
    lax.fori_loop(0, NCH, scat, 0)

    pltpu.sync_copy(adj_v, adj_hbm.at[wid])
    plsc.subcore_barrier()

    @pl.when(sid < 15)
    def _():
        pltpu.sync_copy(acc_sh.at[pl.ds(sid * RPT, RPT)],
                        deg_hbm.at[cid, pl.ds(sid * RPT, RPT)])

    @pl.when(sid == 15)
    def _():
        pltpu.sync_copy(acc_sh.at[pl.ds(15 * RPT, TAIL)],
                        deg_hbm.at[cid, pl.ds(15 * RPT, TAIL)])


@functools.partial(
    pl.kernel,
    out_type=jax.ShapeDtypeStruct((2, N, H), jnp.float32),
    mesh=plsc.VectorSubcoreMesh(**_MESH),
    compiler_params=pltpu.CompilerParams(use_tc_tiling_on_sc=False),
    scratch_types=[
        pltpu.VMEM((EPW,), jnp.int32),
        pltpu.VMEM((NCH, CHUNK), jnp.int32),
        pltpu.VMEM((NBUF, CHUNK, H), jnp.float32),
        pltpu.VMEM((ZR, H), jnp.float32),
        pltpu.VMEM_SHARED((NPAD, H), jnp.float32),
    ] + [pltpu.SemaphoreType.DMA] * NBUF,
)
def _agg(y_hbm, ei_hbm, adj_hbm, out_hbm,
         frm_v, adj_v, bufs, zero_v, acc_sh, *sems):
    cid = lax.axis_index("c")
    sid = lax.axis_index("s")
    wid = cid * 16 + sid
    z16 = jnp.zeros((16,), jnp.float32)

    def zrow(i, _):
        zero_v[i, :] = z16
        return 0
    lax.fori_loop(0, ZR, zrow, 0)

    for t in range(RPT // ZR):
        pltpu.sync_copy(zero_v, acc_sh.at[pl.ds(sid * RPT + t * ZR, ZR)])
    pltpu.sync_copy(ei_hbm.at[0, pl.ds(wid * EPW, EPW)], frm_v)
    pltpu.sync_copy(adj_hbm.at[wid], adj_v)
    plsc.subcore_barrier()

    def idx(j):
        return frm_v.at[pl.ds(j * CHUNK, CHUNK)]

    for b in range(NBUF):
        pltpu.async_copy(y_hbm.at[idx(b)], bufs.at[b], sems[b])

    def ring(g, _):
        j0 = g * NBUF
        for b in range(NBUF):
            pltpu.make_async_copy(
                y_hbm.at[idx(j0 + b)], bufs.at[b], sems[b]).wait()
            pltpu.sync_copy(bufs.at[b], acc_sh.at[adj_v.at[j0 + b]], add=True)
            pltpu.async_copy(
                y_hbm.at[idx(j0 + NBUF + b)], bufs.at[b], sems[b])
        return 0
    lax.fori_loop(0, NRING - 1, ring, 0)

    j0 = (NRING - 1) * NBUF
    for b in range(NBUF):
        pltpu.make_async_copy(
            y_hbm.at[idx(j0 + b)], bufs.at[b], sems[b]).wait()
        pltpu.sync_copy(bufs.at[b], acc_sh.at[adj_v.at[j0 + b]], add=True)

    plsc.subcore_barrier()

    @pl.when(sid < 15)
    def _():
        pltpu.sync_copy(acc_sh.at[pl.ds(sid * RPT, RPT)],
                        out_hbm.at[cid, pl.ds(sid * RPT, RPT)])

    @pl.when(sid == 15)
    def _():
        pltpu.sync_copy(acc_sh.at[pl.ds(15 * RPT, TAIL)],
                        out_hbm.at[cid, pl.ds(15 * RPT, TAIL)])


BLK = 1000


def _tc1_body(deg_ref, x_ref, w_ref, dis_ref, y_ref):
    deg = deg_ref[:, 0] + deg_ref[:, 1] + 1.0
    dis = lax.rsqrt(deg)
    y = jnp.dot(x_ref[...], w_ref[...], preferred_element_type=jnp.float32)
    dis_ref[...] = dis[:, None]
    y_ref[...] = y * dis[:, None]


def _tc2_body(dis_ref, p_ref, y1_ref, b1_ref, z_ref):
    agg = p_ref[0] + p_ref[1] + y1_ref[...]
    h = jnp.maximum(agg * dis_ref[...] + b1_ref[...], 0.0)
    z_ref[...] = h * dis_ref[...]


def _tc3_body(dis_ref, q_ref, z_ref, b2_ref, w2_ref, out_ref):
    agg = (q_ref[0] + q_ref[1] + z_ref[...]) * dis_ref[...]
    logits = jnp.dot(agg, w2_ref[...],
                     preferred_element_type=jnp.float32) + b2_ref[...]
    m = jnp.max(logits, axis=1, keepdims=True)
    s = jnp.log(jnp.sum(jnp.exp(logits - m), axis=1, keepdims=True))
    out_ref[...] = logits - m - s


def _tc1(deg2, x, w1t):
    return pl.pallas_call(
        _tc1_body,
        grid=(N // BLK,),
        in_specs=[
            pl.BlockSpec((BLK, 2), lambda i: (i, 0)),
            pl.BlockSpec((BLK, F), lambda i: (i, 0)),
            pl.BlockSpec((F, H), lambda i: (0, 0)),
        ],
        out_specs=[
            pl.BlockSpec((BLK, 1), lambda i: (i, 0)),
            pl.BlockSpec((BLK, H), lambda i: (i, 0)),
        ],
        out_shape=[
            jax.ShapeDtypeStruct((N, 1), jnp.float32),
            jax.ShapeDtypeStruct((N, H), jnp.float32),
        ],
    )(deg2, x, w1t)


def _tc2(dis, p1, y1, b1):
    return pl.pallas_call(
        _tc2_body,
        grid=(N // BLK,),
        in_specs=[
            pl.BlockSpec((BLK, 1), lambda i: (i, 0)),
            pl.BlockSpec((2, BLK, H), lambda i: (0, i, 0)),
            pl.BlockSpec((BLK, H), lambda i: (i, 0)),
            pl.BlockSpec((1, H), lambda i: (0, 0)),
        ],
        out_specs=pl.BlockSpec((BLK, H), lambda i: (i, 0)),
        out_shape=jax.ShapeDtypeStruct((N, H), jnp.float32),
    )(dis, p1, y1, b1)


def _tc3(dis, p2, z, b2, w2t):
    return pl.pallas_call(
        _tc3_body,
        grid=(N // BLK,),
        in_specs=[
            pl.BlockSpec((BLK, 1), lambda i: (i, 0)),
            pl.BlockSpec((2, BLK, H), lambda i: (0, i, 0)),
            pl.BlockSpec((BLK, H), lambda i: (i, 0)),
            pl.BlockSpec((1, L), lambda i: (0, 0)),
            pl.BlockSpec((H, L), lambda i: (0, 0)),
        ],
        out_specs=pl.BlockSpec((BLK, L), lambda i: (i, 0)),
        out_shape=jax.ShapeDtypeStruct((N, L), jnp.float32),
    )(dis, p2, z, b2, w2t)


def kernel(x, edge_index, W1, b1, W2, b2):
    ei = edge_index.astype(jnp.int32)

    deg_p, adj = _deg_adj(ei)
    dis, y1 = _tc1(deg_p.T, x, W1.T)
    p1 = _agg(y1, ei, adj)
    z = _tc2(dis, p1, y1, b1.reshape(1, H))
    p2 = _agg(z, ei, adj)
    return _tc3(dis, p2, z, b2.reshape(1, L), W2.T)


# speedup vs baseline: 65.0585x; 1.1224x over previous
import functools

import jax
import jax.numpy as jnp
from jax import lax
from jax.experimental import pallas as pl
from jax.experimental.pallas import tpu as pltpu
from jax.experimental.pallas import tpu_sc as plsc

N = 10000
E = 320000
F = 128
H = 16
L = 64

NPAD = 10240
NW = 32
EPW = E // NW
CHUNK = 80
NCH = EPW // CHUNK
RPT = NPAD // 16
NBUF = 5
NRING = NCH // NBUF
ZR = 40
TAIL = N - 15 * RPT

_MESH = dict(core_axis_name="c", subcore_axis_name="s")


@functools.partial(
    pl.kernel,
    out_type=[
        jax.ShapeDtypeStruct((2, N), jnp.float32),
        jax.ShapeDtypeStruct((NW, NCH, CHUNK), jnp.int32),
    ],
    mesh=plsc.VectorSubcoreMesh(**_MESH),
    compiler_params=pltpu.CompilerParams(use_tc_tiling_on_sc=False),
    scratch_types=[
        pltpu.VMEM((EPW,), jnp.int32),
        pltpu.VMEM((EPW,), jnp.int32),
        pltpu.VMEM((NCH, CHUNK), jnp.int32),
        pltpu.VMEM((CHUNK,), jnp.float32),
        pltpu.VMEM((RPT,), jnp.float32),
        pltpu.VMEM_SHARED((NPAD,), jnp.float32),
    ],
)
def _deg_adj(ei_hbm, deg_hbm, adj_hbm,
             frm_v, to_v, adj_v, ones_v, zero_v, acc_sh):
    cid = lax.axis_index("c")
    sid = lax.axis_index("s")
    wid = cid * 16 + sid
    z16 = jnp.zeros((16,), jnp.float32)
    o16 = jnp.ones((16,), jnp.float32)

    def fill_z(i, _):
        zero_v[pl.ds(i * 16, 16)] = z16
        return 0
    lax.fori_loop(0, RPT // 16, fill_z, 0)

    def fill_o(i, _):
        ones_v[pl.ds(i * 16, 16)] = o16
        return 0
    lax.fori_loop(0, CHUNK // 16, fill_o, 0)

    pltpu.sync_copy(zero_v, acc_sh.at[pl.ds(sid * RPT, RPT)])
    pltpu.sync_copy(ei_hbm.at[0, pl.ds(wid * EPW, EPW)], frm_v)
    pltpu.sync_copy(ei_hbm.at[1, pl.ds(wid * EPW, EPW)], to_v)

    def chunk(j, _):
        dump = N + ((wid * NCH + j) * 7) % (NPAD - N)

        def vec(k, _):
            f16 = frm_v[pl.ds(j * CHUNK + k * 16, 16)]
            t16 = to_v[pl.ds(j * CHUNK + k * 16, 16)]
            adj_v[j, pl.ds(k * 16, 16)] = jnp.where(f16 != t16, t16, dump)
            return 0
        lax.fori_loop(0, CHUNK // 16, vec, 0)
        return 0
    lax.fori_loop(0, NCH, chunk, 0)

    plsc.subcore_barrier()

    def scat(j, _):
        pltpu.sync_copy(ones_v, acc_sh.at[adj_v.at[j]], add=True)
        return 0
    lax.fori_loop(0, NCH, scat, 0)

    pltpu.sync_copy(adj_v, adj_hbm.at[wid])
    plsc.subcore_barrier()

    @pl.when(sid < 15)
    def _():
        pltpu.sync_copy(acc_sh.at[pl.ds(sid * RPT, RPT)],
                        deg_hbm.at[cid, pl.ds(sid * RPT, RPT)])

    @pl.when(sid == 15)
    def _():
        pltpu.sync_copy(acc_sh.at[pl.ds(15 * RPT, TAIL)],
                        deg_hbm.at[cid, pl.ds(15 * RPT, TAIL)])


@functools.partial(
    pl.kernel,
    out_type=jax.ShapeDtypeStruct((2, N, H), jnp.float32),
    mesh=plsc.VectorSubcoreMesh(**_MESH),
    compiler_params=pltpu.CompilerParams(use_tc_tiling_on_sc=False),
    scratch_types=[
        pltpu.VMEM((EPW,), jnp.int32),
        pltpu.VMEM((NCH, CHUNK), jnp.int32),
        pltpu.VMEM((NBUF, CHUNK, H), jnp.float32),
        pltpu.VMEM((ZR, H), jnp.float32),
        pltpu.VMEM_SHARED((NPAD, H), jnp.float32),
    ] + [pltpu.SemaphoreType.DMA] * NBUF,
)
def _agg(y_hbm, ei_hbm, adj_hbm, out_hbm,
         frm_v, adj_v, bufs, zero_v, acc_sh, *sems):
    cid = lax.axis_index("c")
    sid = lax.axis_index("s")
    wid = cid * 16 + sid
    z16 = jnp.zeros((16,), jnp.float32)

    def zrow(i, _):
        zero_v[i, :] = z16
        return 0
    lax.fori_loop(0, ZR, zrow, 0)

    for t in range(RPT // ZR):
        pltpu.sync_copy(zero_v, acc_sh.at[pl.ds(sid * RPT + t * ZR, ZR)])
    pltpu.sync_copy(ei_hbm.at[0, pl.ds(wid * EPW, EPW)], frm_v)
    pltpu.sync_copy(adj_hbm.at[wid], adj_v)
    plsc.subcore_barrier()

    def idx(j):
        return frm_v.at[pl.ds(j * CHUNK, CHUNK)]

    for b in range(NBUF):
        pltpu.async_copy(y_hbm.at[idx(b)], bufs.at[b], sems[b])

    def ring(g, _):
        j0 = g * NBUF
        for b in range(NBUF):
            pltpu.make_async_copy(
                y_hbm.at[idx(j0 + b)], bufs.at[b], sems[b]).wait()
            pltpu.sync_copy(bufs.at[b], acc_sh.at[adj_v.at[j0 + b]], add=True)
            pltpu.async_copy(
                y_hbm.at[idx(j0 + NBUF + b)], bufs.at[b], sems[b])
        return 0
    lax.fori_loop(0, NRING - 1, ring, 0)

    j0 = (NRING - 1) * NBUF
    for b in range(NBUF):
        pltpu.make_async_copy(
            y_hbm.at[idx(j0 + b)], bufs.at[b], sems[b]).wait()
        pltpu.sync_copy(bufs.at[b], acc_sh.at[adj_v.at[j0 + b]], add=True)

    plsc.subcore_barrier()

    @pl.when(sid < 15)
    def _():
        pltpu.sync_copy(acc_sh.at[pl.ds(sid * RPT, RPT)],
                        out_hbm.at[cid, pl.ds(sid * RPT, RPT)])

    @pl.when(sid == 15)
    def _():
        pltpu.sync_copy(acc_sh.at[pl.ds(15 * RPT, TAIL)],
                        out_hbm.at[cid, pl.ds(15 * RPT, TAIL)])


@functools.partial(
    pl.kernel,
    out_type=jax.ShapeDtypeStruct((2, N, H), jnp.float32),
    mesh=plsc.VectorSubcoreMesh(**_MESH),
    compiler_params=pltpu.CompilerParams(use_tc_tiling_on_sc=False),
    scratch_types=[
        pltpu.VMEM((EPW,), jnp.int32),
        pltpu.VMEM((NCH, CHUNK), jnp.int32),
        pltpu.VMEM((NBUF, CHUNK, H), jnp.float32),
        pltpu.VMEM((RPT, H), jnp.float32),
        pltpu.VMEM((RPT, H), jnp.float32),
        pltpu.VMEM((RPT, H), jnp.float32),
        pltpu.VMEM((RPT, H), jnp.float32),
        pltpu.VMEM((RPT, H), jnp.float32),
        pltpu.VMEM((16,), jnp.float32),
        pltpu.VMEM((ZR, H), jnp.float32),
        pltpu.VMEM_SHARED((NPAD, H), jnp.float32),
        pltpu.VMEM_SHARED((NPAD, H), jnp.float32),
    ] + [pltpu.SemaphoreType.DMA] * NBUF,
)
def _agg2(p1_hbm, y1_hbm, disb_hbm, b1_hbm, ei_hbm, adj_hbm, out_hbm,
          frm_v, adj_v, bufs, pa_v, pb_v, y_v, z_v, db_v, b1_v, zero_v,
          zsp_sh, acc_sh, *sems):
    cid = lax.axis_index("c")
    sid = lax.axis_index("s")
    wid = cid * 16 + sid
    base = sid * RPT
    z16 = jnp.zeros((16,), jnp.float32)

    pltpu.sync_copy(b1_hbm, b1_v)
    b1vec = b1_v[:]

    def zrow(i, _):
        zero_v[i, :] = z16
        return 0
    lax.fori_loop(0, ZR, zrow, 0)

    def phase1(rn):
        pltpu.sync_copy(p1_hbm.at[0, pl.ds(base, rn)], pa_v.at[pl.ds(0, rn)])
        pltpu.sync_copy(p1_hbm.at[1, pl.ds(base, rn)], pb_v.at[pl.ds(0, rn)])
        pltpu.sync_copy(y1_hbm.at[pl.ds(base, rn)], y_v.at[pl.ds(0, rn)])
        pltpu.sync_copy(disb_hbm.at[pl.ds(base, rn)], db_v.at[pl.ds(0, rn)])

        def row(r, _):
            dbr = db_v[r, :]
            p16 = pa_v[r, :] + pb_v[r, :] + y_v[r, :]
            h = jnp.maximum(p16 * dbr + b1vec, 0.0)
            z_v[r, :] = h * dbr
            return 0
        lax.fori_loop(0, rn, row, 0)

        pltpu.sync_copy(z_v.at[pl.ds(0, rn)], zsp_sh.at[pl.ds(base, rn)])

        @pl.when(cid == 0)
        def _():
            pltpu.sync_copy(z_v.at[pl.ds(0, rn)], acc_sh.at[pl.ds(base, rn)])

        @pl.when(cid == 1)
        def _():
            for t in range(rn // ZR):
                pltpu.sync_copy(zero_v, acc_sh.at[pl.ds(base + t * ZR, ZR)])

    @pl.when(sid < 15)
    def _():
        phase1(RPT)

    @pl.when(sid == 15)
    def _():
        phase1(TAIL)

    pltpu.sync_copy(ei_hbm.at[0, pl.ds(wid * EPW, EPW)], frm_v)
    pltpu.sync_copy(adj_hbm.at[wid], adj_v)
    plsc.subcore_barrier()

    def idx(j):
        return frm_v.at[pl.ds(j * CHUNK, CHUNK)]

    for b in range(NBUF):
        pltpu.async_copy(zsp_sh.at[idx(b)], bufs.at[b], sems[b])

    def ring(g, _):
        j0 = g * NBUF
        for b in range(NBUF):
            pltpu.make_async_copy(
                zsp_sh.at[idx(j0 + b)], bufs.at[b], sems[b]).wait()
            pltpu.sync_copy(bufs.at[b], acc_sh.at[adj_v.at[j0 + b]], add=True)
            pltpu.async_copy(
                zsp_sh.at[idx(j0 + NBUF + b)], bufs.at[b], sems[b])
        return 0
    lax.fori_loop(0, NRING - 1, ring, 0)

    j0 = (NRING - 1) * NBUF
    for b in range(NBUF):
        pltpu.make_async_copy(
            zsp_sh.at[idx(j0 + b)], bufs.at[b], sems[b]).wait()
        pltpu.sync_copy(bufs.at[b], acc_sh.at[adj_v.at[j0 + b]], add=True)

    plsc.subcore_barrier()

    @pl.when(sid < 15)
    def _():
        pltpu.sync_copy(acc_sh.at[pl.ds(sid * RPT, RPT)],
                        out_hbm.at[cid, pl.ds(sid * RPT, RPT)])

    @pl.when(sid == 15)
    def _():
        pltpu.sync_copy(acc_sh.at[pl.ds(15 * RPT, TAIL)],
                        out_hbm.at[cid, pl.ds(15 * RPT, TAIL)])


BLK = 1000


def _tc1_body(deg_ref, x_ref, w_ref, dis_ref, disb_ref, y_ref):
    deg = deg_ref[:, 0] + deg_ref[:, 1] + 1.0
    dis = lax.rsqrt(deg)
    y = jnp.dot(x_ref[...], w_ref[...], preferred_element_type=jnp.float32)
    dis_ref[...] = dis[:, None]
    disb_ref[...] = jnp.broadcast_to(dis[:, None], disb_ref.shape)
    y_ref[...] = y * dis[:, None]


def _tc3_body(dis_ref, q_ref, b2_ref, w2_ref, out_ref):
    agg = (q_ref[0] + q_ref[1]) * dis_ref[...]
    logits = jnp.dot(agg, w2_ref[...],
                     preferred_element_type=jnp.float32) + b2_ref[...]
    m = jnp.max(logits, axis=1, keepdims=True)
    s = jnp.log(jnp.sum(jnp.exp(logits - m), axis=1, keepdims=True))
    out_ref[...] = logits - m - s


def _tc1(deg2, x, w1t):
    return pl.pallas_call(
        _tc1_body,
        grid=(N // BLK,),
        in_specs=[
            pl.BlockSpec((BLK, 2), lambda i: (i, 0)),
            pl.BlockSpec((BLK, F), lambda i: (i, 0)),
            pl.BlockSpec((F, H), lambda i: (0, 0)),
        ],
        out_specs=[
            pl.BlockSpec((BLK, 1), lambda i: (i, 0)),
            pl.BlockSpec((BLK, H), lambda i: (i, 0)),
            pl.BlockSpec((BLK, H), lambda i: (i, 0)),
        ],
        out_shape=[
            jax.ShapeDtypeStruct((N, 1), jnp.float32),
            jax.ShapeDtypeStruct((N, H), jnp.float32),
            jax.ShapeDtypeStruct((N, H), jnp.float32),
        ],
    )(deg2, x, w1t)


def _tc3(dis, p2, b2, w2t):
    return pl.pallas_call(
        _tc3_body,
        grid=(N // BLK,),
        in_specs=[
            pl.BlockSpec((BLK, 1), lambda i: (i, 0)),
            pl.BlockSpec((2, BLK, H), lambda i: (0, i, 0)),
            pl.BlockSpec((1, L), lambda i: (0, 0)),
            pl.BlockSpec((H, L), lambda i: (0, 0)),
        ],
        out_specs=pl.BlockSpec((BLK, L), lambda i: (i, 0)),
        out_shape=jax.ShapeDtypeStruct((N, L), jnp.float32),
    )(dis, p2, b2, w2t)


def kernel(x, edge_index, W1, b1, W2, b2):
    ei = edge_index.astype(jnp.int32)

    deg_p, adj = _deg_adj(ei)
    dis, disb, y1 = _tc1(deg_p.T, x, W1.T)
    p1 = _agg(y1, ei, adj)
    q = _agg2(p1, y1, disb, b1, ei, adj)
    return _tc3(dis, q, b2.reshape(1, L), W2.T)


# speedup vs baseline: 71.9696x; 1.1062x over previous
import functools

import jax
import jax.numpy as jnp
from jax import lax
from jax.experimental import pallas as pl
from jax.experimental.pallas import tpu as pltpu
from jax.experimental.pallas import tpu_sc as plsc

N = 10000
E = 320000
F = 128
H = 16
L = 64

NPAD = 10240
NW = 32
EPW = E // NW
CHUNK = 80
NCH = EPW // CHUNK
RPT = NPAD // 16
NBUF = 5
NRING = NCH // NBUF
ZR = 40
TAIL = N - 15 * RPT

_MESH = dict(core_axis_name="c", subcore_axis_name="s")


@functools.partial(
    pl.kernel,
    out_type=[
        jax.ShapeDtypeStruct((2, N), jnp.float32),
        jax.ShapeDtypeStruct((NW, NCH, CHUNK), jnp.int32),
    ],
    mesh=plsc.VectorSubcoreMesh(**_MESH),
    compiler_params=pltpu.CompilerParams(use_tc_tiling_on_sc=False),
    scratch_types=[
        pltpu.VMEM((EPW,), jnp.int32),
        pltpu.VMEM((EPW,), jnp.int32),
        pltpu.VMEM((NCH, CHUNK), jnp.int32),
        pltpu.VMEM((CHUNK,), jnp.float32),
        pltpu.VMEM((RPT,), jnp.float32),
        pltpu.VMEM_SHARED((NPAD,), jnp.float32),
        pltpu.SemaphoreType.DMA,
    ],
)
def _deg_adj(ei_hbm, deg_hbm, adj_hbm,
             frm_v, to_v, adj_v, ones_v, zero_v, acc_sh, dsem):
    cid = lax.axis_index("c")
    sid = lax.axis_index("s")
    wid = cid * 16 + sid
    z16 = jnp.zeros((16,), jnp.float32)
    o16 = jnp.ones((16,), jnp.float32)

    def fill_z(i, _):
        zero_v[pl.ds(i * 16, 16)] = z16
        return 0
    lax.fori_loop(0, RPT // 16, fill_z, 0)

    def fill_o(i, _):
        ones_v[pl.ds(i * 16, 16)] = o16
        return 0
    lax.fori_loop(0, CHUNK // 16, fill_o, 0)

    pltpu.sync_copy(zero_v, acc_sh.at[pl.ds(sid * RPT, RPT)])
    pltpu.sync_copy(ei_hbm.at[0, pl.ds(wid * EPW, EPW)], frm_v)
    pltpu.sync_copy(ei_hbm.at[1, pl.ds(wid * EPW, EPW)], to_v)
    plsc.subcore_barrier()

    def chunk(j, _):
        dump = N + ((wid * NCH + j) * 7) % (NPAD - N)

        def vec(k, _):
            f16 = frm_v[pl.ds(j * CHUNK + k * 16, 16)]
            t16 = to_v[pl.ds(j * CHUNK + k * 16, 16)]
            adj_v[j, pl.ds(k * 16, 16)] = jnp.where(f16 != t16, t16, dump)
            return 0
        lax.fori_loop(0, CHUNK // 16, vec, 0)
        pltpu.async_copy(ones_v, acc_sh.at[adj_v.at[j]], dsem, add=True)
        return 0
    lax.fori_loop(0, NCH, chunk, 0)

    pltpu.sync_copy(adj_v, adj_hbm.at[wid])

    def drain(j, _):
        pltpu.make_async_copy(ones_v, acc_sh.at[adj_v.at[j]], dsem).wait()
        return 0
    lax.fori_loop(0, NCH, drain, 0)
    plsc.subcore_barrier()

    @pl.when(sid < 15)
    def _():
        pltpu.sync_copy(acc_sh.at[pl.ds(sid * RPT, RPT)],
                        deg_hbm.at[cid, pl.ds(sid * RPT, RPT)])

    @pl.when(sid == 15)
    def _():
        pltpu.sync_copy(acc_sh.at[pl.ds(15 * RPT, TAIL)],
                        deg_hbm.at[cid, pl.ds(15 * RPT, TAIL)])


@functools.partial(
    pl.kernel,
    out_type=jax.ShapeDtypeStruct((2, N, H), jnp.float32),
    mesh=plsc.VectorSubcoreMesh(**_MESH),
    compiler_params=pltpu.CompilerParams(use_tc_tiling_on_sc=False),
    scratch_types=[
        pltpu.VMEM((EPW,), jnp.int32),
        pltpu.VMEM((NCH, CHUNK), jnp.int32),
        pltpu.VMEM((NBUF, CHUNK, H), jnp.float32),
        pltpu.VMEM((ZR, H), jnp.float32),
        pltpu.VMEM_SHARED((NPAD, H), jnp.float32),
    ] + [pltpu.SemaphoreType.DMA] * (2 * NBUF),
)
def _agg(y_hbm, ei_hbm, adj_hbm, out_hbm,
         frm_v, adj_v, bufs, zero_v, acc_sh, *sems):
    cid = lax.axis_index("c")
    sid = lax.axis_index("s")
    wid = cid * 16 + sid
    z16 = jnp.zeros((16,), jnp.float32)

    def zrow(i, _):
        zero_v[i, :] = z16
        return 0
    lax.fori_loop(0, ZR, zrow, 0)

    for t in range(RPT // ZR):
        pltpu.sync_copy(zero_v, acc_sh.at[pl.ds(sid * RPT + t * ZR, ZR)])
    pltpu.sync_copy(ei_hbm.at[0, pl.ds(wid * EPW, EPW)], frm_v)
    pltpu.sync_copy(adj_hbm.at[wid], adj_v)
    plsc.subcore_barrier()

    def idx(j):
        return frm_v.at[pl.ds(j * CHUNK, CHUNK)]

    gsems, ssems = sems[:NBUF], sems[NBUF:]
    for b in range(NBUF):
        pltpu.async_copy(y_hbm.at[idx(b)], bufs.at[b], gsems[b])

    def ring(g, _):
        j0 = g * NBUF
        for b in range(NBUF):
            pltpu.make_async_copy(
                y_hbm.at[idx(j0 + b)], bufs.at[b], gsems[b]).wait()
            pltpu.async_copy(bufs.at[b], acc_sh.at[adj_v.at[j0 + b]],
                             ssems[b], add=True)
            if b > 0:
                pltpu.make_async_copy(
                    bufs.at[b - 1], acc_sh.at[adj_v.at[j0 + b - 1]],
                    ssems[b - 1]).wait()
                pltpu.async_copy(
                    y_hbm.at[idx(j0 + NBUF + b - 1)], bufs.at[b - 1],
                    gsems[b - 1])
        pltpu.make_async_copy(
            bufs.at[NBUF - 1], acc_sh.at[adj_v.at[j0 + NBUF - 1]],
            ssems[NBUF - 1]).wait()
        pltpu.async_copy(
            y_hbm.at[idx(j0 + 2 * NBUF - 1)], bufs.at[NBUF - 1],
            gsems[NBUF - 1])
        return 0
    lax.fori_loop(0, NRING - 1, ring, 0)

    j0 = (NRING - 1) * NBUF
    for b in range(NBUF):
        pltpu.make_async_copy(
            y_hbm.at[idx(j0 + b)], bufs.at[b], gsems[b]).wait()
        pltpu.async_copy(bufs.at[b], acc_sh.at[adj_v.at[j0 + b]],
                         ssems[b], add=True)
    for b in range(NBUF):
        pltpu.make_async_copy(
            bufs.at[b], acc_sh.at[adj_v.at[j0 + b]], ssems[b]).wait()

    plsc.subcore_barrier()

    @pl.when(sid < 15)
    def _():
        pltpu.sync_copy(acc_sh.at[pl.ds(sid * RPT, RPT)],
                        out_hbm.at[cid, pl.ds(sid * RPT, RPT)])

    @pl.when(sid == 15)
    def _():
        pltpu.sync_copy(acc_sh.at[pl.ds(15 * RPT, TAIL)],
                        out_hbm.at[cid, pl.ds(15 * RPT, TAIL)])


@functools.partial(
    pl.kernel,
    out_type=jax.ShapeDtypeStruct((2, N, H), jnp.float32),
    mesh=plsc.VectorSubcoreMesh(**_MESH),
    compiler_params=pltpu.CompilerParams(use_tc_tiling_on_sc=False),
    scratch_types=[
        pltpu.VMEM((EPW,), jnp.int32),
        pltpu.VMEM((NCH, CHUNK), jnp.int32),
        pltpu.VMEM((NBUF, CHUNK, H), jnp.float32),
        pltpu.VMEM((RPT, H), jnp.float32),
        pltpu.VMEM((RPT, H), jnp.float32),
        pltpu.VMEM((RPT, H), jnp.float32),
        pltpu.VMEM((RPT, H), jnp.float32),
        pltpu.VMEM((RPT, H), jnp.float32),
        pltpu.VMEM((16,), jnp.float32),
        pltpu.VMEM((ZR, H), jnp.float32),
        pltpu.VMEM_SHARED((NPAD, H), jnp.float32),
        pltpu.VMEM_SHARED((NPAD, H), jnp.float32),
    ] + [pltpu.SemaphoreType.DMA] * (2 * NBUF),
)
def _agg2(p1_hbm, y1_hbm, disb_hbm, b1_hbm, ei_hbm, adj_hbm, out_hbm,
          frm_v, adj_v, bufs, pa_v, pb_v, y_v, z_v, db_v, b1_v, zero_v,
          zsp_sh, acc_sh, *sems):
    cid = lax.axis_index("c")
    sid = lax.axis_index("s")
    wid = cid * 16 + sid
    base = sid * RPT
    z16 = jnp.zeros((16,), jnp.float32)

    pltpu.sync_copy(b1_hbm, b1_v)
    b1vec = b1_v[:]

    def zrow(i, _):
        zero_v[i, :] = z16
        return 0
    lax.fori_loop(0, ZR, zrow, 0)

    def phase1(rn):
        pltpu.sync_copy(p1_hbm.at[0, pl.ds(base, rn)], pa_v.at[pl.ds(0, rn)])
        pltpu.sync_copy(p1_hbm.at[1, pl.ds(base, rn)], pb_v.at[pl.ds(0, rn)])
        pltpu.sync_copy(y1_hbm.at[pl.ds(base, rn)], y_v.at[pl.ds(0, rn)])
        pltpu.sync_copy(disb_hbm.at[pl.ds(base, rn)], db_v.at[pl.ds(0, rn)])

        def row(r, _):
            dbr = db_v[r, :]
            p16 = pa_v[r, :] + pb_v[r, :] + y_v[r, :]
            h = jnp.maximum(p16 * dbr + b1vec, 0.0)
            z_v[r, :] = h * dbr
            return 0
        lax.fori_loop(0, rn, row, 0)

        pltpu.sync_copy(z_v.at[pl.ds(0, rn)], zsp_sh.at[pl.ds(base, rn)])

        @pl.when(cid == 0)
        def _():
            pltpu.sync_copy(z_v.at[pl.ds(0, rn)], acc_sh.at[pl.ds(base, rn)])

        @pl.when(cid == 1)
        def _():
            for t in range(rn // ZR):
                pltpu.sync_copy(zero_v, acc_sh.at[pl.ds(base + t * ZR, ZR)])

    @pl.when(sid < 15)
    def _():
        phase1(RPT)

    @pl.when(sid == 15)
    def _():
        phase1(TAIL)

    pltpu.sync_copy(ei_hbm.at[0, pl.ds(wid * EPW, EPW)], frm_v)
    pltpu.sync_copy(adj_hbm.at[wid], adj_v)
    plsc.subcore_barrier()

    def idx(j):
        return frm_v.at[pl.ds(j * CHUNK, CHUNK)]

    gsems, ssems = sems[:NBUF], sems[NBUF:]
    for b in range(NBUF):
        pltpu.async_copy(zsp_sh.at[idx(b)], bufs.at[b], gsems[b])

    def ring(g, _):
        j0 = g * NBUF
        for b in range(NBUF):
            pltpu.make_async_copy(
                zsp_sh.at[idx(j0 + b)], bufs.at[b], gsems[b]).wait()
            pltpu.async_copy(bufs.at[b], acc_sh.at[adj_v.at[j0 + b]],
                             ssems[b], add=True)
            if b > 0:
                pltpu.make_async_copy(
                    bufs.at[b - 1], acc_sh.at[adj_v.at[j0 + b - 1]],
                    ssems[b - 1]).wait()
                pltpu.async_copy(
                    zsp_sh.at[idx(j0 + NBUF + b - 1)], bufs.at[b - 1],
                    gsems[b - 1])
        pltpu.make_async_copy(
            bufs.at[NBUF - 1], acc_sh.at[adj_v.at[j0 + NBUF - 1]],
            ssems[NBUF - 1]).wait()
        pltpu.async_copy(
            zsp_sh.at[idx(j0 + 2 * NBUF - 1)], bufs.at[NBUF - 1],
            gsems[NBUF - 1])
        return 0
    lax.fori_loop(0, NRING - 1, ring, 0)

    j0 = (NRING - 1) * NBUF
    for b in range(NBUF):
        pltpu.make_async_copy(
            zsp_sh.at[idx(j0 + b)], bufs.at[b], gsems[b]).wait()
        pltpu.async_copy(bufs.at[b], acc_sh.at[adj_v.at[j0 + b]],
                         ssems[b], add=True)
    for b in range(NBUF):
        pltpu.make_async_copy(
            bufs.at[b], acc_sh.at[adj_v.at[j0 + b]], ssems[b]).wait()

    plsc.subcore_barrier()

    @pl.when(sid < 15)
    def _():
        pltpu.sync_copy(acc_sh.at[pl.ds(sid * RPT, RPT)],
                        out_hbm.at[cid, pl.ds(sid * RPT, RPT)])

    @pl.when(sid == 15)
    def _():
        pltpu.sync_copy(acc_sh.at[pl.ds(15 * RPT, TAIL)],
                        out_hbm.at[cid, pl.ds(15 * RPT, TAIL)])


BLK = 1000


def _tc1_body(deg_ref, x_ref, w_ref, dis_ref, disb_ref, y_ref):
    deg = deg_ref[:, 0] + deg_ref[:, 1] + 1.0
    dis = lax.rsqrt(deg)
    y = jnp.dot(x_ref[...], w_ref[...], preferred_element_type=jnp.float32)
    dis_ref[...] = dis[:, None]
    disb_ref[...] = jnp.broadcast_to(dis[:, None], disb_ref.shape)
    y_ref[...] = y * dis[:, None]


def _tc3_body(dis_ref, q_ref, b2_ref, w2_ref, out_ref):
    agg = (q_ref[0] + q_ref[1]) * dis_ref[...]
    logits = jnp.dot(agg, w2_ref[...],
                     preferred_element_type=jnp.float32) + b2_ref[...]
    m = jnp.max(logits, axis=1, keepdims=True)
    s = jnp.log(jnp.sum(jnp.exp(logits - m), axis=1, keepdims=True))
    out_ref[...] = logits - m - s


def _tc1(deg2, x, w1t):
    return pl.pallas_call(
        _tc1_body,
        grid=(N // BLK,),
        in_specs=[
            pl.BlockSpec((BLK, 2), lambda i: (i, 0)),
            pl.BlockSpec((BLK, F), lambda i: (i, 0)),
            pl.BlockSpec((F, H), lambda i: (0, 0)),
        ],
        out_specs=[
            pl.BlockSpec((BLK, 1), lambda i: (i, 0)),
            pl.BlockSpec((BLK, H), lambda i: (i, 0)),
            pl.BlockSpec((BLK, H), lambda i: (i, 0)),
        ],
        out_shape=[
            jax.ShapeDtypeStruct((N, 1), jnp.float32),
            jax.ShapeDtypeStruct((N, H), jnp.float32),
            jax.ShapeDtypeStruct((N, H), jnp.float32),
        ],
    )(deg2, x, w1t)


def _tc3(dis, p2, b2, w2t):
    return pl.pallas_call(
        _tc3_body,
        grid=(N // BLK,),
        in_specs=[
            pl.BlockSpec((BLK, 1), lambda i: (i, 0)),
            pl.BlockSpec((2, BLK, H), lambda i: (0, i, 0)),
            pl.BlockSpec((1, L), lambda i: (0, 0)),
            pl.BlockSpec((H, L), lambda i: (0, 0)),
        ],
        out_specs=pl.BlockSpec((BLK, L), lambda i: (i, 0)),
        out_shape=jax.ShapeDtypeStruct((N, L), jnp.float32),
    )(dis, p2, b2, w2t)


def kernel(x, edge_index, W1, b1, W2, b2):
    ei = edge_index.astype(jnp.int32)

    deg_p, adj = _deg_adj(ei)
    dis, disb, y1 = _tc1(deg_p.T, x, W1.T)
    p1 = _agg(y1, ei, adj)
    q = _agg2(p1, y1, disb, b1, ei, adj)
    return _tc3(dis, q, b2.reshape(1, L), W2.T)
